# baseline (speedup 1.0000x reference)
import functools

import jax
import jax.numpy as jnp
from jax import lax
from jax.experimental import pallas as pl
from jax.experimental.pallas import tpu as pltpu
from jax.experimental.pallas import tpu_sc as plsc

_RB = 16
_NB = 4


def _gather_fn(nc, ns, b_per_w, table_hbm, idx_hbm, out_hbm, idx_v, rows_v, sem):
    wid = lax.axis_index("s") * nc + lax.axis_index("c")
    base = wid * b_per_w
    pltpu.sync_copy(idx_hbm.at[pl.ds(base, b_per_w)], idx_v)
    pltpu.async_copy(table_hbm.at[idx_v], rows_v, sem).wait()
    pltpu.sync_copy(rows_v, out_hbm.at[pl.ds(base, b_per_w)])


def _sc_gather(table, idx_flat):
    info = plsc.get_sparse_core_info()
    nc, ns = info.num_cores, info.num_subcores
    nw = nc * ns
    b_total, d = idx_flat.shape[0], table.shape[1]
    b_per_w = b_total // nw
    mesh = plsc.VectorSubcoreMesh(core_axis_name="c", subcore_axis_name="s")
    kern = functools.partial(
        pl.kernel,
        mesh=mesh,
        out_type=jax.ShapeDtypeStruct((b_total, d), jnp.float32),
        scratch_types=[
            pltpu.VMEM((b_per_w,), jnp.int32),
            pltpu.VMEM((b_per_w, d), jnp.float32),
            pltpu.SemaphoreType.DMA,
        ],
        compiler_params=pltpu.CompilerParams(use_tc_tiling_on_sc=False),
    )(functools.partial(_gather_fn, nc, ns, b_per_w))
    return kern(table, idx_flat)


def _proj_body(ng, h_ref, w_ref, b_ref, out_hbm, *scratch):
    accs, sems = scratch[:_NB], scratch[_NB:]
    g = pl.program_id(0)
    v = out_hbm.shape[1]

    for k in range(_NB):
        tile = g * _NB + k

        @pl.when(g >= 1)
        def _wait_prev(k=k, tile=tile):
            pltpu.make_async_copy(
                accs[k],
                out_hbm.at[pl.ds((tile - _NB) * _RB, _RB), :],
                sems[k],
            ).wait()

        accs[k][...] = (
            jnp.dot(
                h_ref[pl.ds(k * _RB, _RB), :],
                w_ref[...],
                preferred_element_type=jnp.float32,
            )
            + b_ref[...]
        )
        pltpu.make_async_copy(
            accs[k], out_hbm.at[pl.ds(tile * _RB, _RB), :], sems[k]
        ).start()

    @pl.when(g == ng - 1)
    def _drain():
        for k in range(_NB):
            tile = (ng - 1) * _NB + k
            pltpu.make_async_copy(
                accs[k], out_hbm.at[pl.ds(tile * _RB, _RB), :], sems[k]
            ).wait()


def _tc_project(hidden, proj_w, proj_b2d):
    r, h = hidden.shape
    v = proj_w.shape[1]
    ng = r // (_RB * _NB)
    return pl.pallas_call(
        functools.partial(_proj_body, ng),
        grid=(ng,),
        in_specs=[
            pl.BlockSpec((_RB * _NB, h), lambda g: (g, 0)),
            pl.BlockSpec((h, v), lambda g: (0, 0)),
            pl.BlockSpec((1, v), lambda g: (0, 0)),
        ],
        out_specs=pl.BlockSpec(memory_space=pl.ANY),
        out_shape=jax.ShapeDtypeStruct((r, v), jnp.float32),
        scratch_shapes=(
            [pltpu.VMEM((_RB, v), jnp.float32) for _ in range(_NB)]
            + [pltpu.SemaphoreType.DMA for _ in range(_NB)]
        ),
    )(hidden, proj_w, proj_b2d)


def kernel(input_ids, emb_table, proj_w, proj_b):
    b, q = input_ids.shape
    v = proj_w.shape[1]
    idx_flat = input_ids.reshape(b * q).astype(jnp.int32)
    hidden = _sc_gather(emb_table, idx_flat)
    logits = _tc_project(hidden, proj_w, proj_b.reshape(1, v))
    return logits.reshape(b, q, v)
